# initial kernel scaffold (unmeasured)
import jax
import jax.numpy as jnp
from jax import lax
from jax.experimental import pallas as pl
from jax.experimental.pallas import tpu as pltpu

NZ = 4
M_GLOBAL = 8192
M_CHUNK = M_GLOBAL // NZ
D = 2048


def kernel(partial, gamma):
    partial = partial.reshape(M_GLOBAL, D)
    gamma = gamma.reshape(1, D)

    def body(partial_ref, gamma_ref, out_ref,
             send_buf, local_buf, recv_buf, send_sem, recv_sems, local_sem):
        my_x = lax.axis_index("x")
        my_y = lax.axis_index("y")
        my_z = lax.axis_index("z")
        right = (my_z + 1) % NZ
        left = (my_z - 1) % NZ

        barrier = pltpu.get_barrier_semaphore()
        for nbr in (left, right):
            pl.semaphore_signal(
                barrier, inc=1,
                device_id=(my_x, my_y, nbr),
                device_id_type=pl.DeviceIdType.MESH,
            )
        pl.semaphore_wait(barrier, 2)

        c0 = (my_z - 1) % NZ
        cp = pltpu.make_async_copy(
            partial_ref.at[pl.ds(c0 * M_CHUNK, M_CHUNK), :], send_buf, local_sem)
        cp.start()
        cp.wait()

        for s in range(NZ - 1):
            rdma = pltpu.make_async_remote_copy(
                src_ref=send_buf,
                dst_ref=recv_buf.at[s],
                send_sem=send_sem,
                recv_sem=recv_sems.at[s],
                device_id=(my_x, my_y, right),
                device_id_type=pl.DeviceIdType.MESH,
            )
            rdma.start()
            c_recv = (my_z - 2 - s) % NZ
            cp = pltpu.make_async_copy(
                partial_ref.at[pl.ds(c_recv * M_CHUNK, M_CHUNK), :],
                local_buf, local_sem)
            cp.start()
            rdma.wait()
            cp.wait()
            if s < NZ - 2:
                send_buf[...] = local_buf[...] + recv_buf[s]
            else:
                y = local_buf[...] + recv_buf[s]
                ms = jnp.mean(y * y, axis=-1, keepdims=True)
                out_ref[...] = y * lax.rsqrt(ms + 1e-6) * gamma_ref[...]

    return pl.pallas_call(
        body,
        out_shape=jax.ShapeDtypeStruct((M_CHUNK, D), jnp.float32),
        in_specs=[
            pl.BlockSpec(memory_space=pltpu.ANY),
            pl.BlockSpec(memory_space=pltpu.VMEM),
        ],
        out_specs=pl.BlockSpec(memory_space=pltpu.VMEM),
        scratch_shapes=[
            pltpu.VMEM((M_CHUNK, D), jnp.float32),
            pltpu.VMEM((M_CHUNK, D), jnp.float32),
            pltpu.VMEM((NZ - 1, M_CHUNK, D), jnp.float32),
            pltpu.SemaphoreType.DMA,
            pltpu.SemaphoreType.DMA((NZ - 1,)),
            pltpu.SemaphoreType.DMA,
        ],
        compiler_params=pltpu.CompilerParams(collective_id=0),
    )(partial, gamma)


# baseline (device time: 611559 ns/iter reference)
import jax
import jax.numpy as jnp
from jax import lax
from jax.experimental import pallas as pl
from jax.experimental.pallas import tpu as pltpu

NZ = 4
M_GLOBAL = 8192
M_CHUNK = M_GLOBAL // NZ
D = 2048
TR = 1024
NT = M_CHUNK // TR


def kernel(partial, gamma):
    partial = partial.reshape(M_GLOBAL, D)
    gamma = gamma.reshape(1, D)

    def body(partial_ref, gamma_ref, out_ref, send_buf, recv_buf,
             va, vb, send_sem, recv_sems, sem_a, sem_b, sem_out):
        my_x = lax.axis_index("x")
        my_y = lax.axis_index("y")
        my_z = lax.axis_index("z")
        right = (my_z + 1) % NZ
        left = (my_z - 1) % NZ

        barrier = pltpu.get_barrier_semaphore()
        for nbr in (left, right):
            pl.semaphore_signal(
                barrier, inc=1,
                device_id=(my_x, my_y, nbr),
                device_id_type=pl.DeviceIdType.MESH,
            )
        pl.semaphore_wait(barrier, 2)

        c0 = (my_z - 1) % NZ

        for s in range(NZ - 1):
            if s == 0:
                src = partial_ref.at[pl.ds(c0 * M_CHUNK, M_CHUNK), :]
            else:
                src = send_buf
            rdma = pltpu.make_async_remote_copy(
                src_ref=src,
                dst_ref=recv_buf.at[s],
                send_sem=send_sem,
                recv_sem=recv_sems.at[s],
                device_id=(my_x, my_y, right),
                device_id_type=pl.DeviceIdType.MESH,
            )
            rdma.start()
            rdma.wait()

            c_recv = (my_z - 2 - s) % NZ
            for t in range(NT):
                cpa = pltpu.make_async_copy(
                    partial_ref.at[pl.ds(c_recv * M_CHUNK + t * TR, TR), :],
                    va, sem_a)
                cpb = pltpu.make_async_copy(
                    recv_buf.at[s, pl.ds(t * TR, TR), :], vb, sem_b)
                cpa.start()
                cpb.start()
                cpa.wait()
                cpb.wait()
                acc = va[...] + vb[...]
                if s < NZ - 2:
                    va[...] = acc
                    cpo = pltpu.make_async_copy(
                        va, send_buf.at[pl.ds(t * TR, TR), :], sem_out)
                    cpo.start()
                    cpo.wait()
                else:
                    ms = jnp.mean(acc * acc, axis=-1, keepdims=True)
                    out_ref[pl.ds(t * TR, TR), :] = (
                        acc * lax.rsqrt(ms + 1e-6) * gamma_ref[...])

    out, _, _ = pl.pallas_call(
        body,
        out_shape=[
            jax.ShapeDtypeStruct((M_CHUNK, D), jnp.float32),
            jax.ShapeDtypeStruct((M_CHUNK, D), jnp.float32),
            jax.ShapeDtypeStruct((NZ - 1, M_CHUNK, D), jnp.float32),
        ],
        in_specs=[
            pl.BlockSpec(memory_space=pl.ANY),
            pl.BlockSpec(memory_space=pltpu.VMEM),
        ],
        out_specs=[
            pl.BlockSpec(memory_space=pltpu.VMEM),
            pl.BlockSpec(memory_space=pl.ANY),
            pl.BlockSpec(memory_space=pl.ANY),
        ],
        scratch_shapes=[
            pltpu.VMEM((TR, D), jnp.float32),
            pltpu.VMEM((TR, D), jnp.float32),
            pltpu.SemaphoreType.DMA,
            pltpu.SemaphoreType.DMA((NZ - 1,)),
            pltpu.SemaphoreType.DMA,
            pltpu.SemaphoreType.DMA,
            pltpu.SemaphoreType.DMA,
        ],
        compiler_params=pltpu.CompilerParams(collective_id=0),
    )(partial, gamma)
    return out


# device time: 567661 ns/iter; 1.0773x vs baseline; 1.0773x over previous
import jax
import jax.numpy as jnp
from jax import lax
from jax.experimental import pallas as pl
from jax.experimental.pallas import tpu as pltpu

NZ = 4
M_GLOBAL = 8192
M_CHUNK = M_GLOBAL // NZ
D = 2048
DH = D // 2
TR = 1024
NT = M_CHUNK // TR


def kernel(partial, gamma):
    partial = partial.reshape(M_GLOBAL, D)
    gamma = gamma.reshape(1, D)

    def body(partial_ref, gamma_ref, out_ref, recv_cw, recv_ccw,
             vsend_cw, vsend_ccw, va, vb, vc, vd,
             ssem_cw, ssem_ccw, rsem_cw, rsem_ccw,
             sem_a, sem_b, sem_c, sem_d):
        my_x = lax.axis_index("x")
        my_y = lax.axis_index("y")
        my_z = lax.axis_index("z")
        right = (my_z + 1) % NZ
        left = (my_z - 1) % NZ

        barrier = pltpu.get_barrier_semaphore()
        for nbr in (left, right):
            pl.semaphore_signal(
                barrier, inc=1,
                device_id=(my_x, my_y, nbr),
                device_id_type=pl.DeviceIdType.MESH,
            )
        pl.semaphore_wait(barrier, 2)

        dirs = [
            dict(tgt=right, sgn=-1, col0=0, recv=recv_cw, vsend=vsend_cw,
                 ssem=ssem_cw, rsem=rsem_cw, vl=va, vr=vb, sl=sem_a, sr=sem_b),
            dict(tgt=left, sgn=+1, col0=DH, recv=recv_ccw, vsend=vsend_ccw,
                 ssem=ssem_ccw, rsem=rsem_ccw, vl=vc, vr=vd, sl=sem_c,
                 sr=sem_d),
        ]

        def chunk_rows(c, t):
            return pl.ds(c * M_CHUNK + t * TR, TR)

        rdmas = {}

        for di, dd in enumerate(dirs):
            c_send = (my_z + dd["sgn"]) % NZ
            for t in range(NT):
                r = pltpu.make_async_remote_copy(
                    src_ref=partial_ref.at[chunk_rows(c_send, t),
                                           pl.ds(dd["col0"], DH)],
                    dst_ref=dd["recv"].at[0, pl.ds(t * TR, TR), :],
                    send_sem=dd["ssem"].at[t],
                    recv_sem=dd["rsem"].at[0, t],
                    device_id=(my_x, my_y, dd["tgt"]),
                    device_id_type=pl.DeviceIdType.MESH,
                )
                r.start()
                rdmas[(di, 0, t)] = r

        for s in range(NZ - 2):
            for t in range(NT):
                for di, dd in enumerate(dirs):
                    c_recv = (my_z + dd["sgn"] * (s + 2)) % NZ
                    cpl = pltpu.make_async_copy(
                        partial_ref.at[chunk_rows(c_recv, t),
                                       pl.ds(dd["col0"], DH)],
                        dd["vl"], dd["sl"])
                    cpl.start()
                    r = rdmas[(di, s, t)]
                    r.wait_recv()
                    cpr = pltpu.make_async_copy(
                        dd["recv"].at[s, pl.ds(t * TR, TR), :],
                        dd["vr"], dd["sr"])
                    cpr.start()
                    cpl.wait()
                    cpr.wait()
                    r.wait_send()
                    dd["vsend"][t, :, :] = dd["vl"][...] + dd["vr"][...]
                    r2 = pltpu.make_async_remote_copy(
                        src_ref=dd["vsend"].at[t],
                        dst_ref=dd["recv"].at[s + 1, pl.ds(t * TR, TR), :],
                        send_sem=dd["ssem"].at[t],
                        recv_sem=dd["rsem"].at[s + 1, t],
                        device_id=(my_x, my_y, dd["tgt"]),
                        device_id_type=pl.DeviceIdType.MESH,
                    )
                    r2.start()
                    rdmas[(di, s + 1, t)] = r2

        s = NZ - 2
        for t in range(NT):
            cps = []
            for dd in dirs:
                cpl = pltpu.make_async_copy(
                    partial_ref.at[chunk_rows(my_z, t), pl.ds(dd["col0"], DH)],
                    dd["vl"], dd["sl"])
                cpl.start()
                cps.append(cpl)
            for di, dd in enumerate(dirs):
                rdmas[(di, s, t)].wait_recv()
                cpr = pltpu.make_async_copy(
                    dd["recv"].at[s, pl.ds(t * TR, TR), :], dd["vr"], dd["sr"])
                cpr.start()
                cps.append(cpr)
            for cp in cps:
                cp.wait()
            y0 = va[...] + vb[...]
            y1 = vc[...] + vd[...]
            ms = (jnp.sum(y0 * y0, axis=-1, keepdims=True)
                  + jnp.sum(y1 * y1, axis=-1, keepdims=True)) / D
            scale = lax.rsqrt(ms + 1e-6)
            rows = pl.ds(t * TR, TR)
            out_ref[rows, pl.ds(0, DH)] = y0 * scale * gamma_ref[:, 0:DH]
            out_ref[rows, pl.ds(DH, DH)] = y1 * scale * gamma_ref[:, DH:D]

        for di in range(2):
            for t in range(NT):
                rdmas[(di, s, t)].wait_send()

    out, _, _ = pl.pallas_call(
        body,
        out_shape=[
            jax.ShapeDtypeStruct((M_CHUNK, D), jnp.float32),
            jax.ShapeDtypeStruct((NZ - 1, M_CHUNK, DH), jnp.float32),
            jax.ShapeDtypeStruct((NZ - 1, M_CHUNK, DH), jnp.float32),
        ],
        in_specs=[
            pl.BlockSpec(memory_space=pl.ANY),
            pl.BlockSpec(memory_space=pltpu.VMEM),
        ],
        out_specs=[
            pl.BlockSpec(memory_space=pltpu.VMEM),
            pl.BlockSpec(memory_space=pl.ANY),
            pl.BlockSpec(memory_space=pl.ANY),
        ],
        scratch_shapes=[
            pltpu.VMEM((NT, TR, DH), jnp.float32),
            pltpu.VMEM((NT, TR, DH), jnp.float32),
            pltpu.VMEM((TR, DH), jnp.float32),
            pltpu.VMEM((TR, DH), jnp.float32),
            pltpu.VMEM((TR, DH), jnp.float32),
            pltpu.VMEM((TR, DH), jnp.float32),
            pltpu.SemaphoreType.DMA((NT,)),
            pltpu.SemaphoreType.DMA((NT,)),
            pltpu.SemaphoreType.DMA((NZ - 1, NT)),
            pltpu.SemaphoreType.DMA((NZ - 1, NT)),
            pltpu.SemaphoreType.DMA,
            pltpu.SemaphoreType.DMA,
            pltpu.SemaphoreType.DMA,
            pltpu.SemaphoreType.DMA,
        ],
        compiler_params=pltpu.CompilerParams(
            collective_id=0, vmem_limit_bytes=60 * 1024 * 1024),
    )(partial, gamma)
    return out


# device time: 188749 ns/iter; 3.2401x vs baseline; 3.0075x over previous
import jax
import jax.numpy as jnp
from jax import lax
from jax.experimental import pallas as pl
from jax.experimental.pallas import tpu as pltpu

NZ = 4
NR = 8
M_GLOBAL = 8192
M_CHUNK = M_GLOBAL // NZ
D = 2048
DB = D // NR
DBH = DB // 2


def _snake_coords(rr):
    x = jnp.where(rr < 4, 0, 1)
    y = jnp.where(rr < 4, rr, 7 - rr)
    return x, y


def kernel(partial, gamma):
    partial = partial.reshape(M_GLOBAL, D)
    gamma = gamma.reshape(1, D)

    def body(partial_ref, gamma_ref, out_ref,
             recv_cw, recv_ccw, vsend_cw, vsend_ccw, va, vc,
             zssem_cw, zssem_ccw, zrsem_cw, zrsem_ccw,
             agssem, agrsem, sem_a, sem_c):
        my_x = lax.axis_index("x")
        my_y = lax.axis_index("y")
        my_z = lax.axis_index("z")
        zright = (my_z + 1) % NZ
        zleft = (my_z - 1) % NZ
        r = jnp.where(my_x == 0, my_y, 7 - my_y)
        rnext = (r + 1) % NR
        rprev = (r - 1) % NR
        nx_x, nx_y = _snake_coords(rnext)
        pv_x, pv_y = _snake_coords(rprev)
        dcol0 = r * DB

        barrier = pltpu.get_barrier_semaphore()
        for dev in ((my_x, my_y, zleft), (my_x, my_y, zright),
                    (nx_x, nx_y, my_z), (pv_x, pv_y, my_z)):
            pl.semaphore_signal(
                barrier, inc=1, device_id=dev,
                device_id_type=pl.DeviceIdType.MESH)
        pl.semaphore_wait(barrier, 4)

        dirs = [
            dict(tgt=zright, sgn=-1, col0=dcol0, recv=recv_cw,
                 vsend=vsend_cw, ssem=zssem_cw, rsem=zrsem_cw,
                 vl=va, sl=sem_a),
            dict(tgt=zleft, sgn=+1, col0=dcol0 + DBH, recv=recv_ccw,
                 vsend=vsend_ccw, ssem=zssem_ccw, rsem=zrsem_ccw,
                 vl=vc, sl=sem_c),
        ]

        rd = {}
        for di, dd in enumerate(dirs):
            c_send = (my_z + dd["sgn"]) % NZ
            rdma = pltpu.make_async_remote_copy(
                src_ref=partial_ref.at[pl.ds(c_send * M_CHUNK, M_CHUNK),
                                       pl.ds(dd["col0"], DBH)],
                dst_ref=dd["recv"].at[0],
                send_sem=dd["ssem"],
                recv_sem=dd["rsem"].at[0],
                device_id=(my_x, my_y, dd["tgt"]),
                device_id_type=pl.DeviceIdType.MESH,
            )
            rdma.start()
            rd[(di, 0)] = rdma

        for s in range(NZ - 1):
            for di, dd in enumerate(dirs):
                c_recv = (my_z + dd["sgn"] * (s + 2)) % NZ
                cpl = pltpu.make_async_copy(
                    partial_ref.at[pl.ds(c_recv * M_CHUNK, M_CHUNK),
                                   pl.ds(dd["col0"], DBH)],
                    dd["vl"], dd["sl"])
                cpl.start()
                rd[(di, s)].wait_recv()
                cpl.wait()
                if s < NZ - 2:
                    rd[(di, s)].wait_send()
                    dd["vsend"][...] = dd["vl"][...] + dd["recv"][s]
                    r2 = pltpu.make_async_remote_copy(
                        src_ref=dd["vsend"],
                        dst_ref=dd["recv"].at[s + 1],
                        send_sem=dd["ssem"],
                        recv_sem=dd["rsem"].at[s + 1],
                        device_id=(my_x, my_y, dd["tgt"]),
                        device_id_type=pl.DeviceIdType.MESH,
                    )
                    r2.start()
                    rd[(di, s + 1)] = r2
                else:
                    rd[(di, s)].wait_send()
                    out_ref[:, pl.ds(dd["col0"], DBH)] = (
                        dd["vl"][...] + dd["recv"][s])

        agdirs = [
            dict(tgt=(nx_x, nx_y, my_z), sgn=-1, off=0),
            dict(tgt=(pv_x, pv_y, my_z), sgn=+1, off=DBH),
        ]
        ag = {}
        for h in range(NR - 1):
            for di, dd in enumerate(agdirs):
                o_send = (r + dd["sgn"] * h) % NR
                if h > 0:
                    ag[(di, h - 1)].wait_recv()
                    ag[(di, h - 1)].wait_send()
                rdma = pltpu.make_async_remote_copy(
                    src_ref=out_ref.at[:, pl.ds(o_send * DB + dd["off"], DBH)],
                    dst_ref=out_ref.at[:, pl.ds(o_send * DB + dd["off"], DBH)],
                    send_sem=agssem.at[di],
                    recv_sem=agrsem.at[di, h],
                    device_id=dd["tgt"],
                    device_id_type=pl.DeviceIdType.MESH,
                )
                rdma.start()
                ag[(di, h)] = rdma
        for di in range(2):
            ag[(di, NR - 2)].wait_recv()
            ag[(di, NR - 2)].wait_send()

        for tb in range(4):
            rows = pl.ds(tb * (M_CHUNK // 4), M_CHUNK // 4)
            y = out_ref[rows, :]
            ms = jnp.mean(y * y, axis=-1, keepdims=True)
            out_ref[rows, :] = y * lax.rsqrt(ms + 1e-6) * gamma_ref[...]

    return pl.pallas_call(
        body,
        out_shape=jax.ShapeDtypeStruct((M_CHUNK, D), jnp.float32),
        in_specs=[
            pl.BlockSpec(memory_space=pl.ANY),
            pl.BlockSpec(memory_space=pltpu.VMEM),
        ],
        out_specs=pl.BlockSpec(memory_space=pltpu.VMEM),
        scratch_shapes=[
            pltpu.VMEM((NZ - 1, M_CHUNK, DBH), jnp.float32),
            pltpu.VMEM((NZ - 1, M_CHUNK, DBH), jnp.float32),
            pltpu.VMEM((M_CHUNK, DBH), jnp.float32),
            pltpu.VMEM((M_CHUNK, DBH), jnp.float32),
            pltpu.VMEM((M_CHUNK, DBH), jnp.float32),
            pltpu.VMEM((M_CHUNK, DBH), jnp.float32),
            pltpu.SemaphoreType.DMA,
            pltpu.SemaphoreType.DMA,
            pltpu.SemaphoreType.DMA((NZ - 1,)),
            pltpu.SemaphoreType.DMA((NZ - 1,)),
            pltpu.SemaphoreType.DMA((2,)),
            pltpu.SemaphoreType.DMA((2, NR - 1)),
            pltpu.SemaphoreType.DMA,
            pltpu.SemaphoreType.DMA,
        ],
        compiler_params=pltpu.CompilerParams(
            collective_id=0, vmem_limit_bytes=60 * 1024 * 1024),
    )(partial, gamma)
